# Initial kernel scaffold; baseline (speedup 1.0000x reference)
import jax
import jax.numpy as jnp
from jax.experimental import pallas as pl


def kernel(x, edge_index, W1, b1, W2, b2):
    raise NotImplementedError("write your pallas kernel here")


# speedup vs baseline: 31.6351x; 31.6351x over previous
import functools

import jax
import jax.numpy as jnp
from jax import lax
from jax.experimental import pallas as pl
from jax.experimental.pallas import tpu as pltpu
from jax.experimental.pallas import tpu_sc as plsc

_N = 10000
_E = 320000
_IN = 128
_HID = 64
_OUT = 64

_NC = 2
_NS = 16
_L = 16
_CHUNK = 128
_NCHUNK = 2560
_EPAD = _NCHUNK * _CHUNK
_CPT = _NCHUNK // (_NC * _NS)
_NPAD = 10112
_RPT = _NPAD // _NS
_RSTAGE = _NPAD // _NS

_ROWBLK = 632


def _mesh():
    return plsc.VectorSubcoreMesh(core_axis_name="c", subcore_axis_name="s")


def _deg_body(dst_hbm, out_hbm, dst_v, ones_v, zeros_v, acc):
    c = lax.axis_index("c")
    s = lax.axis_index("s")
    wid = c * _NS + s

    @pl.loop(0, _CHUNK)
    def _fill(i):
        ones_v[i, pl.ds(0, _L)] = jnp.ones((_L,), jnp.float32)
        zeros_v[i, pl.ds(0, _L)] = jnp.zeros((_L,), jnp.float32)

    row0 = s * _RPT
    nfull = _RPT // _CHUNK
    for b in range(nfull):
        pltpu.sync_copy(zeros_v, acc.at[pl.ds(row0 + b * _CHUNK, _CHUNK)])
    rem = _RPT - nfull * _CHUNK
    pltpu.sync_copy(zeros_v.at[pl.ds(0, rem)],
                    acc.at[pl.ds(row0 + nfull * _CHUNK, rem)])

    pltpu.sync_copy(dst_hbm.at[pl.ds(wid * _CPT, _CPT)], dst_v)
    plsc.subcore_barrier()

    @pl.loop(0, _CPT)
    def _accum(k):
        pltpu.sync_copy(ones_v, acc.at[dst_v.at[k]], add=True)

    plsc.subcore_barrier()
    pltpu.sync_copy(acc.at[pl.ds(row0, _RPT)],
                    out_hbm.at[c, pl.ds(row0, _RPT)])


def _sc_degree(dst2d):
    kfn = functools.partial(
        pl.kernel,
        compiler_params=pltpu.CompilerParams(use_tc_tiling_on_sc=False),
        out_type=jax.ShapeDtypeStruct((_NC, _NPAD, _L), jnp.float32),
        mesh=_mesh(),
        scratch_types=[
            pltpu.VMEM((_CPT, _CHUNK), jnp.int32),
            pltpu.VMEM((_CHUNK, _L), jnp.float32),
            pltpu.VMEM((_CHUNK, _L), jnp.float32),
            pltpu.VMEM_SHARED((_NPAD, _L), jnp.float32),
        ],
    )(_deg_body)
    return kfn(dst2d)


_NBUF = 2


def _gs_body(g_hbm, src_hbm, dst_hbm, out_hbm, src_v, dst_v,
             r0, r1, gs0, gs1, ss0, ss1, zb, g_s, acc):
    rows = (r0, r1)
    gsem = (gs0, gs1)
    ssem = (ss0, ss1)
    c = lax.axis_index("c")
    s = lax.axis_index("s")
    wid = c * _NS + s
    row0 = s * _RPT
    gr0 = s * _RSTAGE

    pltpu.sync_copy(src_hbm.at[pl.ds(wid * _CPT, _CPT)], src_v)
    pltpu.sync_copy(dst_hbm.at[pl.ds(wid * _CPT, _CPT)], dst_v)

    @pl.loop(0, _CHUNK)
    def _fill(i):
        for j in range(_HID // _L):
            zb[i, pl.ds(j * _L, _L)] = jnp.zeros((_L,), jnp.float32)

    nfull = _RPT // _CHUNK
    for b in range(nfull):
        pltpu.sync_copy(zb, acc.at[pl.ds(row0 + b * _CHUNK, _CHUNK)])
    rem = _RPT - nfull * _CHUNK
    pltpu.sync_copy(zb.at[pl.ds(0, rem)],
                    acc.at[pl.ds(row0 + nfull * _CHUNK, rem)])

    pltpu.sync_copy(g_hbm.at[pl.ds(gr0, _RSTAGE)],
                    g_s.at[pl.ds(gr0, _RSTAGE)])
    plsc.subcore_barrier()

    for b in range(_NBUF):
        pltpu.async_copy(g_s.at[src_v.at[b]], rows[b], gsem[b])

    @pl.loop(0, _CPT, step=_NBUF)
    def _edges(k):
        for b in range(_NBUF):
            pltpu.make_async_copy(
                g_s.at[src_v.at[k + b]], rows[b], gsem[b]).wait()
            pltpu.async_copy(rows[b], acc.at[dst_v.at[k + b]], ssem[b],
                             add=True)
        for b in range(_NBUF):
            pltpu.make_async_copy(
                rows[b], acc.at[dst_v.at[k + b]], ssem[b]).wait()

            @pl.when(k + _NBUF + b < _CPT)
            def _prefetch(b=b):
                pltpu.async_copy(
                    g_s.at[src_v.at[k + _NBUF + b]], rows[b], gsem[b])

    plsc.subcore_barrier()
    pltpu.sync_copy(acc.at[pl.ds(row0, _RPT)],
                    out_hbm.at[c, pl.ds(row0, _RPT)])


def _sc_gather_scatter(g, src2d, dst2d):
    kfn = functools.partial(
        pl.kernel,
        compiler_params=pltpu.CompilerParams(use_tc_tiling_on_sc=False),
        out_type=jax.ShapeDtypeStruct((_NC, _NPAD, _HID), jnp.float32),
        mesh=_mesh(),
        scratch_types=[
            pltpu.VMEM((_CPT, _CHUNK), jnp.int32),
            pltpu.VMEM((_CPT, _CHUNK), jnp.int32),
        ] + [pltpu.VMEM((_CHUNK, _HID), jnp.float32)] * _NBUF
          + [pltpu.SemaphoreType.DMA] * (2 * _NBUF)
          + [pltpu.VMEM((_CHUNK, _HID), jnp.float32)]
          + [pltpu.VMEM_SHARED((_NPAD, _HID), jnp.float32)] * 2,
    )(_gs_body)
    return kfn(g, src2d, dst2d)


def _dinv_block(d_ref):
    deg = d_ref[0, :, 0:1] + d_ref[1, :, 0:1] + 1.0
    return lax.rsqrt(deg)


def _tc_matmul_scale(x, degp, W1):
    def body(x_ref, d_ref, w_ref, g_ref):
        dinv = _dinv_block(d_ref)
        g_ref[...] = lax.dot_general(
            x_ref[...] * dinv, w_ref[...], (((1,), (0,)), ((), ())),
            preferred_element_type=jnp.float32,
            precision=lax.Precision.HIGHEST)

    return pl.pallas_call(
        body,
        grid=(_NPAD // _ROWBLK,),
        in_specs=[pl.BlockSpec((_ROWBLK, _IN), lambda i: (i, 0)),
                  pl.BlockSpec((_NC, _ROWBLK, _L), lambda i: (0, i, 0)),
                  pl.BlockSpec((_IN, _HID), lambda i: (0, 0))],
        out_specs=pl.BlockSpec((_ROWBLK, _HID), lambda i: (i, 0)),
        out_shape=jax.ShapeDtypeStruct((_NPAD, _HID), jnp.float32),
    )(x, degp, W1)


def _tc_combine1(tp, g, degp, b1, W2):
    def body(t_ref, g_ref, d_ref, b_ref, w_ref, o_ref):
        dinv = _dinv_block(d_ref)
        t = t_ref[0] + t_ref[1] + g_ref[...]
        z = jnp.maximum(t * dinv + b_ref[...], 0.0)
        h2 = lax.dot_general(
            z, w_ref[...], (((1,), (0,)), ((), ())),
            preferred_element_type=jnp.float32,
            precision=lax.Precision.HIGHEST)
        o_ref[...] = h2 * dinv

    return pl.pallas_call(
        body,
        grid=(_NPAD // _ROWBLK,),
        in_specs=[pl.BlockSpec((_NC, _ROWBLK, _HID), lambda i: (0, i, 0)),
                  pl.BlockSpec((_ROWBLK, _HID), lambda i: (i, 0)),
                  pl.BlockSpec((_NC, _ROWBLK, _L), lambda i: (0, i, 0)),
                  pl.BlockSpec((1, _HID), lambda i: (0, 0)),
                  pl.BlockSpec((_HID, _HID), lambda i: (0, 0))],
        out_specs=pl.BlockSpec((_ROWBLK, _HID), lambda i: (i, 0)),
        out_shape=jax.ShapeDtypeStruct((_NPAD, _HID), jnp.float32),
    )(tp, g, degp, b1, W2)


_FBLK = 2000


def _tc_final(tp, g, degp, b2):
    def body(t_ref, g_ref, d_ref, b_ref, o_ref):
        dinv = _dinv_block(d_ref)
        t = t_ref[0] + t_ref[1] + g_ref[...]
        z = t * dinv + b_ref[...]
        m = jnp.max(z, axis=1, keepdims=True)
        lse = jnp.log(jnp.sum(jnp.exp(z - m), axis=1, keepdims=True)) + m
        o_ref[...] = z - lse

    return pl.pallas_call(
        body,
        grid=(_N // _FBLK,),
        in_specs=[pl.BlockSpec((_NC, _FBLK, _HID), lambda i: (0, i, 0)),
                  pl.BlockSpec((_FBLK, _HID), lambda i: (i, 0)),
                  pl.BlockSpec((_NC, _FBLK, _L), lambda i: (0, i, 0)),
                  pl.BlockSpec((1, _OUT), lambda i: (0, 0))],
        out_specs=pl.BlockSpec((_FBLK, _OUT), lambda i: (i, 0)),
        out_shape=jax.ShapeDtypeStruct((_N, _OUT), jnp.float32),
    )(tp, g, degp, b2)


def kernel(x, edge_index, W1, b1, W2, b2):
    src = edge_index[0].astype(jnp.int32)
    dst = edge_index[1].astype(jnp.int32)
    pad = _EPAD - _E
    src2d = jnp.concatenate(
        [src, jnp.zeros((pad,), jnp.int32)]).reshape(_NCHUNK, _CHUNK)
    dst2d = jnp.concatenate(
        [dst, jnp.full((pad,), _N, jnp.int32)]).reshape(_NCHUNK, _CHUNK)

    degp = _sc_degree(dst2d)
    gh1 = _tc_matmul_scale(x, degp, W1)
    t1 = _sc_gather_scatter(gh1, src2d, dst2d)
    gh2 = _tc_combine1(t1, gh1, degp, b1.reshape(1, _HID), W2)
    t2 = _sc_gather_scatter(gh2, src2d, dst2d)
    return _tc_final(t2, gh2, degp, b2.reshape(1, _OUT))
